# Initial kernel scaffold; baseline (speedup 1.0000x reference)
import jax
import jax.numpy as jnp
from jax.experimental import pallas as pl


def kernel(x, edge_index, edge_index1, edge_index2, W_ln, b_ln, W1, b1, W2, b2, W3, b3):
    raise NotImplementedError("write your pallas kernel here")


# speedup vs baseline: 26.7761x; 26.7761x over previous
import functools

import jax
import jax.numpy as jnp
from jax import lax
from jax.experimental import pallas as pl
from jax.experimental.pallas import tpu as pltpu
from jax.experimental.pallas import tpu_sc as plsc

N = 10000
E = 320000
D = 128
NC = 2
NS = 16
NW = NC * NS
EPW = E // NW
B = 80
STEPS = EPW // B
NPAD = 10240
DSL = NPAD // NS
R = 400
GRID = N // R


def _sc_degree(d3_1, d3_2, d3_3):
    mesh = plsc.VectorSubcoreMesh(core_axis_name="c", subcore_axis_name="s")

    @functools.partial(
        pl.kernel, mesh=mesh,
        compiler_params=pltpu.CompilerParams(use_tc_tiling_on_sc=False),
        out_type=jax.ShapeDtypeStruct((3, NC, NPAD), jnp.float32),
        scratch_types=[
            pltpu.VMEM_SHARED((NPAD,), jnp.float32),
            pltpu.VMEM((DSL,), jnp.float32),
            pltpu.VMEM((B,), jnp.float32),
            pltpu.VMEM((STEPS, B), jnp.int32),
        ],
    )
    def deg_k(d1, d2, d3, out, deg_sh, zbuf, ones, idx):
        cid = lax.axis_index("c")
        tid = lax.axis_index("s")
        w = cid * NS + tid

        def fill_z(j, _):
            zbuf[pl.ds(j * 16, 16)] = jnp.zeros((16,), jnp.float32)
            return 0
        lax.fori_loop(0, DSL // 16, fill_z, 0)
        for k in range(B // 16):
            ones[pl.ds(k * 16, 16)] = jnp.ones((16,), jnp.float32)

        for c, dref in enumerate((d1, d2, d3)):
            pltpu.sync_copy(zbuf, deg_sh.at[pl.ds(tid * DSL, DSL)])
            plsc.subcore_barrier()
            pltpu.sync_copy(dref.at[w], idx)

            def step(j, _):
                pltpu.sync_copy(ones, deg_sh.at[idx.at[j]], add=True)
                return 0
            lax.fori_loop(0, STEPS, step, 0)
            plsc.subcore_barrier()
            pltpu.sync_copy(deg_sh.at[pl.ds(tid * DSL, DSL)],
                            out.at[c, cid, pl.ds(tid * DSL, DSL)])
            plsc.subcore_barrier()

    return deg_k(d3_1, d3_2, d3_3)


def _mm_body(x_ref, w_ref, bln_ref, dega_ref, degb_ref, x0_ref, y1_ref,
             y2_ref, y3_ref, dinv_ref):
    xw = jnp.dot(x_ref[...], w_ref[...], preferred_element_type=jnp.float32)
    deg = dega_ref[...] + degb_ref[...] + 1.0
    dinv = lax.rsqrt(deg)
    dinv_ref[...] = dinv
    x0_ref[...] = xw[:, :D] + bln_ref[...]
    y1_ref[...] = dinv[:, 0:1] * xw[:, D:2 * D]
    y2_ref[...] = dinv[:, 1:2] * xw[:, 2 * D:3 * D]
    y3_ref[...] = dinv[:, 2:3] * xw[:, 3 * D:4 * D]


def _tc_matmul(x, w_cat, b_ln2, dega, degb):
    outs = pl.pallas_call(
        _mm_body,
        grid=(GRID,),
        in_specs=[
            pl.BlockSpec((R, D), lambda r: (r, 0)),
            pl.BlockSpec((D, 4 * D), lambda r: (0, 0)),
            pl.BlockSpec((1, D), lambda r: (0, 0)),
            pl.BlockSpec((R, 3), lambda r: (r, 0)),
            pl.BlockSpec((R, 3), lambda r: (r, 0)),
        ],
        out_specs=[
            pl.BlockSpec((R, D), lambda r: (r, 0)),
            pl.BlockSpec((R, D), lambda r: (r, 0)),
            pl.BlockSpec((R, D), lambda r: (r, 0)),
            pl.BlockSpec((R, D), lambda r: (r, 0)),
            pl.BlockSpec((R, 3), lambda r: (r, 0)),
        ],
        out_shape=[
            jax.ShapeDtypeStruct((N, D), jnp.float32),
            jax.ShapeDtypeStruct((N, D), jnp.float32),
            jax.ShapeDtypeStruct((N, D), jnp.float32),
            jax.ShapeDtypeStruct((N, D), jnp.float32),
            jax.ShapeDtypeStruct((N, 3), jnp.float32),
        ],
    )(x, w_cat, b_ln2, dega, degb)
    return outs


def _sc_scatter(y1, y2, y3, s3_1, d3_1, s3_2, d3_2, s3_3, d3_3):
    mesh = plsc.VectorSubcoreMesh(core_axis_name="c", subcore_axis_name="s")

    @functools.partial(
        pl.kernel, mesh=mesh,
        compiler_params=pltpu.CompilerParams(use_tc_tiling_on_sc=False),
        out_type=jax.ShapeDtypeStruct((3, NC, NPAD, D), jnp.float32),
        scratch_types=[
            pltpu.VMEM_SHARED((NPAD, D), jnp.float32),
            pltpu.VMEM((STEPS, B), jnp.int32),
            pltpu.VMEM((STEPS, B), jnp.int32),
            pltpu.VMEM((B, D), jnp.float32),
            pltpu.VMEM((B, D), jnp.float32),
            pltpu.SemaphoreType.DMA,
        ],
    )
    def scat_k(yy1, yy2, yy3, ss1, dd1, ss2, dd2, ss3, dd3, out,
               acc, sidx, didx, rows, zrows, sem):
        cid = lax.axis_index("c")
        tid = lax.axis_index("s")
        w = cid * NS + tid

        def fill_z(j, _):
            for k in range(D // 16):
                zrows[j, pl.ds(k * 16, 16)] = jnp.zeros((16,), jnp.float32)
            return 0
        lax.fori_loop(0, B, fill_z, 0)

        for c, (yref, sref, dref) in enumerate(
                ((yy1, ss1, dd1), (yy2, ss2, dd2), (yy3, ss3, dd3))):
            for q in range(DSL // B):
                pltpu.sync_copy(zrows, acc.at[pl.ds(tid * DSL + q * B, B)])
            plsc.subcore_barrier()
            pltpu.sync_copy(sref.at[w], sidx)
            pltpu.sync_copy(dref.at[w], didx)

            def step(j, _):
                pltpu.async_copy(yref.at[sidx.at[j]], rows, sem).wait()
                pltpu.sync_copy(rows, acc.at[didx.at[j]], add=True)
                return 0
            lax.fori_loop(0, STEPS, step, 0)
            plsc.subcore_barrier()
            pltpu.sync_copy(acc.at[pl.ds(tid * DSL, DSL)],
                            out.at[c, cid, pl.ds(tid * DSL, DSL)])
            plsc.subcore_barrier()

    return scat_k(y1, y2, y3, s3_1, d3_1, s3_2, d3_2, s3_3, d3_3)


def _fin_body(accp_ref, y1_ref, y2_ref, y3_ref, dinv_ref, bcat_ref,
              x1_ref, x2_ref, x3_ref):
    d0 = dinv_ref[:, 0:1]
    d1 = dinv_ref[:, 1:2]
    d2 = dinv_ref[:, 2:3]
    x1_ref[...] = d0 * (accp_ref[0, 0] + accp_ref[0, 1] + y1_ref[...]) \
        + bcat_ref[0:1, :]
    x2_ref[...] = d1 * (accp_ref[1, 0] + accp_ref[1, 1] + y2_ref[...]) \
        + bcat_ref[1:2, :]
    x3_ref[...] = d2 * (accp_ref[2, 0] + accp_ref[2, 1] + y3_ref[...]) \
        + bcat_ref[2:3, :]


def _tc_final(accp, y1, y2, y3, dinv3, b_cat):
    return pl.pallas_call(
        _fin_body,
        grid=(GRID,),
        in_specs=[
            pl.BlockSpec((3, NC, R, D), lambda r: (0, 0, r, 0)),
            pl.BlockSpec((R, D), lambda r: (r, 0)),
            pl.BlockSpec((R, D), lambda r: (r, 0)),
            pl.BlockSpec((R, D), lambda r: (r, 0)),
            pl.BlockSpec((R, 3), lambda r: (r, 0)),
            pl.BlockSpec((3, D), lambda r: (0, 0)),
        ],
        out_specs=[
            pl.BlockSpec((R, D), lambda r: (r, 0)),
            pl.BlockSpec((R, D), lambda r: (r, 0)),
            pl.BlockSpec((R, D), lambda r: (r, 0)),
        ],
        out_shape=[
            jax.ShapeDtypeStruct((N, D), jnp.float32),
            jax.ShapeDtypeStruct((N, D), jnp.float32),
            jax.ShapeDtypeStruct((N, D), jnp.float32),
        ],
    )(accp, y1, y2, y3, dinv3, b_cat)


def kernel(x, edge_index, edge_index1, edge_index2,
           W_ln, b_ln, W1, b1, W2, b2, W3, b3):
    def split(ei):
        e = ei.astype(jnp.int32)
        return (e[0].reshape(NW, STEPS, B), e[1].reshape(NW, STEPS, B))

    s1, d1 = split(edge_index)
    s2, d2 = split(edge_index1)
    s3, d3 = split(edge_index2)

    degp = _sc_degree(d1, d2, d3)
    dega = jnp.transpose(degp[:, 0, :N])
    degb = jnp.transpose(degp[:, 1, :N])
    w_cat = jnp.concatenate([W_ln, W1, W2, W3], axis=1)
    x0, y1, y2, y3, dinv3 = _tc_matmul(x, w_cat, b_ln.reshape(1, D),
                                       dega, degb)
    accp = _sc_scatter(y1, y2, y3, s1, d1, s2, d2, s3, d3)
    b_cat = jnp.stack([b1, b2, b3])
    x1, x2, x3 = _tc_final(accp, y1, y2, y3, dinv3, b_cat)
    return (x0, x1, x2, x3)


# speedup vs baseline: 33.2882x; 1.2432x over previous
import functools

import jax
import jax.numpy as jnp
from jax import lax
from jax.experimental import pallas as pl
from jax.experimental.pallas import tpu as pltpu
from jax.experimental.pallas import tpu_sc as plsc

N = 10000
E = 320000
D = 128
NC = 2
NS = 16
NW = NC * NS
EPW = E // NW
B = 80
STEPS = EPW // B
NPAD = 10240
DSL = NPAD // NS
R = 400
GRID = N // R


def _sc_degree(d3_1, d3_2, d3_3):
    mesh = plsc.VectorSubcoreMesh(core_axis_name="c", subcore_axis_name="s")

    @functools.partial(
        pl.kernel, mesh=mesh,
        compiler_params=pltpu.CompilerParams(use_tc_tiling_on_sc=False),
        out_type=jax.ShapeDtypeStruct((3, NC, NPAD), jnp.float32),
        scratch_types=[
            pltpu.VMEM_SHARED((NPAD,), jnp.float32),
            pltpu.VMEM((DSL,), jnp.float32),
            pltpu.VMEM((B,), jnp.float32),
            pltpu.VMEM((STEPS, B), jnp.int32),
        ],
    )
    def deg_k(d1, d2, d3, out, deg_sh, zbuf, ones, idx):
        cid = lax.axis_index("c")
        tid = lax.axis_index("s")
        w = cid * NS + tid

        def fill_z(j, _):
            zbuf[pl.ds(j * 16, 16)] = jnp.zeros((16,), jnp.float32)
            return 0
        lax.fori_loop(0, DSL // 16, fill_z, 0)
        for k in range(B // 16):
            ones[pl.ds(k * 16, 16)] = jnp.ones((16,), jnp.float32)

        for c, dref in enumerate((d1, d2, d3)):
            pltpu.sync_copy(zbuf, deg_sh.at[pl.ds(tid * DSL, DSL)])
            plsc.subcore_barrier()
            pltpu.sync_copy(dref.at[w], idx)

            def step(j, _):
                pltpu.sync_copy(ones, deg_sh.at[idx.at[j]], add=True)
                return 0
            lax.fori_loop(0, STEPS, step, 0)
            plsc.subcore_barrier()
            pltpu.sync_copy(deg_sh.at[pl.ds(tid * DSL, DSL)],
                            out.at[c, cid, pl.ds(tid * DSL, DSL)])
            plsc.subcore_barrier()

    return deg_k(d3_1, d3_2, d3_3)


def _mm_body(x_ref, w_ref, bln_ref, dega_ref, degb_ref, x0_ref, y1_ref,
             y2_ref, y3_ref, dinv_ref):
    xw = jnp.dot(x_ref[...], w_ref[...], preferred_element_type=jnp.float32)
    deg = dega_ref[...] + degb_ref[...] + 1.0
    dinv = lax.rsqrt(deg)
    dinv_ref[...] = dinv
    x0_ref[...] = xw[:, :D] + bln_ref[...]
    y1_ref[...] = dinv[:, 0:1] * xw[:, D:2 * D]
    y2_ref[...] = dinv[:, 1:2] * xw[:, 2 * D:3 * D]
    y3_ref[...] = dinv[:, 2:3] * xw[:, 3 * D:4 * D]


def _tc_matmul(x, w_cat, b_ln2, dega, degb):
    outs = pl.pallas_call(
        _mm_body,
        grid=(GRID,),
        in_specs=[
            pl.BlockSpec((R, D), lambda r: (r, 0)),
            pl.BlockSpec((D, 4 * D), lambda r: (0, 0)),
            pl.BlockSpec((1, D), lambda r: (0, 0)),
            pl.BlockSpec((R, 3), lambda r: (r, 0)),
            pl.BlockSpec((R, 3), lambda r: (r, 0)),
        ],
        out_specs=[
            pl.BlockSpec((R, D), lambda r: (r, 0)),
            pl.BlockSpec((R, D), lambda r: (r, 0)),
            pl.BlockSpec((R, D), lambda r: (r, 0)),
            pl.BlockSpec((R, D), lambda r: (r, 0)),
            pl.BlockSpec((R, 3), lambda r: (r, 0)),
        ],
        out_shape=[
            jax.ShapeDtypeStruct((N, D), jnp.float32),
            jax.ShapeDtypeStruct((N, D), jnp.float32),
            jax.ShapeDtypeStruct((N, D), jnp.float32),
            jax.ShapeDtypeStruct((N, D), jnp.float32),
            jax.ShapeDtypeStruct((N, 3), jnp.float32),
        ],
    )(x, w_cat, b_ln2, dega, degb)
    return outs


def _sc_scatter(y1, y2, y3, s3_1, d3_1, s3_2, d3_2, s3_3, d3_3):
    mesh = plsc.VectorSubcoreMesh(core_axis_name="c", subcore_axis_name="s")

    @functools.partial(
        pl.kernel, mesh=mesh,
        compiler_params=pltpu.CompilerParams(use_tc_tiling_on_sc=False),
        out_type=jax.ShapeDtypeStruct((3, NC, NPAD, D), jnp.float32),
        scratch_types=[
            pltpu.VMEM_SHARED((NPAD, D), jnp.float32),
            pltpu.VMEM((STEPS, B), jnp.int32),
            pltpu.VMEM((STEPS, B), jnp.int32),
            pltpu.VMEM((B, D), jnp.float32),
            pltpu.VMEM((B, D), jnp.float32),
            pltpu.SemaphoreType.DMA,
            pltpu.SemaphoreType.DMA,
        ],
    )
    def scat_k(yy1, yy2, yy3, ss1, dd1, ss2, dd2, ss3, dd3, out,
               acc, sidx, didx, rows0, rows1, sem0, sem1):
        cid = lax.axis_index("c")
        tid = lax.axis_index("s")
        w = cid * NS + tid

        for c, (yref, sref, dref) in enumerate(
                ((yy1, ss1, dd1), (yy2, ss2, dd2), (yy3, ss3, dd3))):
            def fill_z(j, _):
                for k in range(D // 16):
                    rows0[j, pl.ds(k * 16, 16)] = jnp.zeros((16,),
                                                            jnp.float32)
                return 0
            lax.fori_loop(0, B, fill_z, 0)
            for q in range(DSL // B):
                pltpu.sync_copy(rows0, acc.at[pl.ds(tid * DSL + q * B, B)])
            plsc.subcore_barrier()
            pltpu.sync_copy(sref.at[w], sidx)
            pltpu.sync_copy(dref.at[w], didx)

            pltpu.async_copy(yref.at[sidx.at[0]], rows0, sem0)

            def dstep(i, _):
                j = 2 * i
                pltpu.make_async_copy(yref.at[sidx.at[j]],
                                      rows0, sem0).wait()
                pltpu.async_copy(yref.at[sidx.at[j + 1]], rows1, sem1)
                pltpu.sync_copy(rows0, acc.at[didx.at[j]], add=True)
                pltpu.make_async_copy(yref.at[sidx.at[j + 1]],
                                      rows1, sem1).wait()
                pltpu.async_copy(yref.at[sidx.at[j + 2]], rows0, sem0)
                pltpu.sync_copy(rows1, acc.at[didx.at[j + 1]], add=True)
                return 0
            lax.fori_loop(0, (STEPS - 1) // 2, dstep, 0)
            pltpu.make_async_copy(yref.at[sidx.at[STEPS - 1]],
                                  rows0, sem0).wait()
            pltpu.sync_copy(rows0, acc.at[didx.at[STEPS - 1]], add=True)
            plsc.subcore_barrier()
            pltpu.sync_copy(acc.at[pl.ds(tid * DSL, DSL)],
                            out.at[c, cid, pl.ds(tid * DSL, DSL)])
            plsc.subcore_barrier()

    return scat_k(y1, y2, y3, s3_1, d3_1, s3_2, d3_2, s3_3, d3_3)


def _fin_body(accp_ref, y1_ref, y2_ref, y3_ref, dinv_ref, bcat_ref,
              x1_ref, x2_ref, x3_ref):
    d0 = dinv_ref[:, 0:1]
    d1 = dinv_ref[:, 1:2]
    d2 = dinv_ref[:, 2:3]
    x1_ref[...] = d0 * (accp_ref[0, 0] + accp_ref[0, 1] + y1_ref[...]) \
        + bcat_ref[0:1, :]
    x2_ref[...] = d1 * (accp_ref[1, 0] + accp_ref[1, 1] + y2_ref[...]) \
        + bcat_ref[1:2, :]
    x3_ref[...] = d2 * (accp_ref[2, 0] + accp_ref[2, 1] + y3_ref[...]) \
        + bcat_ref[2:3, :]


def _tc_final(accp, y1, y2, y3, dinv3, b_cat):
    return pl.pallas_call(
        _fin_body,
        grid=(GRID,),
        in_specs=[
            pl.BlockSpec((3, NC, R, D), lambda r: (0, 0, r, 0)),
            pl.BlockSpec((R, D), lambda r: (r, 0)),
            pl.BlockSpec((R, D), lambda r: (r, 0)),
            pl.BlockSpec((R, D), lambda r: (r, 0)),
            pl.BlockSpec((R, 3), lambda r: (r, 0)),
            pl.BlockSpec((3, D), lambda r: (0, 0)),
        ],
        out_specs=[
            pl.BlockSpec((R, D), lambda r: (r, 0)),
            pl.BlockSpec((R, D), lambda r: (r, 0)),
            pl.BlockSpec((R, D), lambda r: (r, 0)),
        ],
        out_shape=[
            jax.ShapeDtypeStruct((N, D), jnp.float32),
            jax.ShapeDtypeStruct((N, D), jnp.float32),
            jax.ShapeDtypeStruct((N, D), jnp.float32),
        ],
    )(accp, y1, y2, y3, dinv3, b_cat)


def kernel(x, edge_index, edge_index1, edge_index2,
           W_ln, b_ln, W1, b1, W2, b2, W3, b3):
    def split(ei):
        e = ei.astype(jnp.int32)
        return (e[0].reshape(NW, STEPS, B), e[1].reshape(NW, STEPS, B))

    s1, d1 = split(edge_index)
    s2, d2 = split(edge_index1)
    s3, d3 = split(edge_index2)

    degp = _sc_degree(d1, d2, d3)
    dega = jnp.transpose(degp[:, 0, :N])
    degb = jnp.transpose(degp[:, 1, :N])
    w_cat = jnp.concatenate([W_ln, W1, W2, W3], axis=1)
    x0, y1, y2, y3, dinv3 = _tc_matmul(x, w_cat, b_ln.reshape(1, D),
                                       dega, degb)
    accp = _sc_scatter(y1, y2, y3, s1, d1, s2, d2, s3, d3)
    b_cat = jnp.stack([b1, b2, b3])
    x1, x2, x3 = _tc_final(accp, y1, y2, y3, dinv3, b_cat)
    return (x0, x1, x2, x3)
